# baseline (speedup 1.0000x reference)
import functools

import jax
import jax.numpy as jnp
from jax import lax
from jax.experimental import pallas as pl
from jax.experimental.pallas import tpu as pltpu
from jax.experimental.pallas import tpu_sc as plsc

H = 768
F = 3072
E = 8
NT = 8192
TBLK = 256
NB = 72
NBP = 80
R = NB * TBLK
TILES = 16
APT = NT // TILES
RPT = R // TILES
W32 = 32
R2 = R // 2
NB2 = NB // 2
RPW = R2 // W32
GCH = 48
TPW = NT // W32
CCH = 16


def _gate_body(x_ref, wg_ref, i1_ref, i2_ref, w1_ref, w2_ref):
    x = x_ref[...]
    logits = jax.lax.dot_general(
        x, wg_ref[...], (((1,), (1,)), ((), ())),
        preferred_element_type=jnp.float32,
    )
    probs = jax.nn.softmax(logits, axis=-1)
    eidx = jax.lax.broadcasted_iota(jnp.int32, probs.shape, 1)
    p1 = jnp.max(probs, axis=-1, keepdims=True)
    first1 = jnp.min(jnp.where(probs == p1, eidx, E), axis=-1, keepdims=True)
    m1 = eidx == first1
    probsb = jnp.where(m1, -jnp.inf, probs)
    p2 = jnp.max(probsb, axis=-1, keepdims=True)
    first2 = jnp.min(jnp.where(probsb == p2, eidx, E), axis=-1, keepdims=True)
    denom = p1 + p2
    i1_ref[...] = first1.astype(jnp.int32).reshape(1, 4, 128)
    i2_ref[...] = first2.astype(jnp.int32).reshape(1, 4, 128)
    w1_ref[...] = (p1 / denom).reshape(1, 4, 128)
    w2_ref[...] = (p2 / denom).reshape(1, 4, 128)


def _gate(tokens, Wg):
    return pl.pallas_call(
        _gate_body,
        grid=(NT // 512,),
        in_specs=[
            pl.BlockSpec((512, H), lambda t: (t, 0)),
            pl.BlockSpec((E, H), lambda t: (0, 0)),
        ],
        out_specs=[
            pl.BlockSpec((1, 4, 128), lambda t: (t, 0, 0)),
            pl.BlockSpec((1, 4, 128), lambda t: (t, 0, 0)),
            pl.BlockSpec((1, 4, 128), lambda t: (t, 0, 0)),
            pl.BlockSpec((1, 4, 128), lambda t: (t, 0, 0)),
        ],
        out_shape=[
            jax.ShapeDtypeStruct((16, 4, 128), jnp.int32),
            jax.ShapeDtypeStruct((16, 4, 128), jnp.int32),
            jax.ShapeDtypeStruct((16, 4, 128), jnp.float32),
            jax.ShapeDtypeStruct((16, 4, 128), jnp.float32),
        ],
    )(tokens, Wg)


def _iota16():
    return jax.lax.iota(jnp.int32, 16)


def _splat(vmem_ref, lane):
    return plsc.load_gather(vmem_ref, [jnp.full((16,), lane, jnp.int32)])


def _route_body(i1_hbm, i2_hbm, w1_hbm, w2_hbm,
                rt_hbm, ws_hbm, fl_hbm, pos0_hbm, pos1_hbm, bexp_hbm, bval_hbm,
                i1_v, i2_v, wv1_v, wv2_v, d1_v, d2_v, tok_v, one_v,
                cntrow_v, cnt_v, ebk_v, base_v, bexp_v, bval_v,
                zz_v, zzf_v,
                cnt_sh, rt_sh, ws_sh, fl_sh):
    cid = lax.axis_index("c")
    sid = lax.axis_index("s")

    @pl.when(cid == 0)
    def _core0():
        pltpu.sync_copy(i1_hbm.at[sid], i1_v)
        pltpu.sync_copy(i2_hbm.at[sid], i2_v)
        pltpu.sync_copy(w1_hbm.at[sid], wv1_v)
        pltpu.sync_copy(w2_hbm.at[sid], wv2_v)

        def hist_arr(arr_ref, acc):
            for r in range(4):
                def hbody(c, acc):
                    v = arr_ref[r, pl.ds(c * 16, 16)]
                    for e in range(E):
                        pop = jnp.sum(jnp.where(v == e, 1, 0))
                        acc = acc + jnp.where(_iota16() == e, pop, 0)
                    return acc
                acc = lax.fori_loop(0, 8, hbody, acc)
            return acc

        cnt = jnp.zeros((16,), jnp.int32)
        cnt = hist_arr(i1_v, cnt)
        cnt = hist_arr(i2_v, cnt)
        cntrow_v[...] = cnt
        pltpu.sync_copy(cntrow_v, cnt_sh.at[pl.ds(sid * 16, 16)])

        def zbody(i, _):
            zz_v[pl.ds(i * 16, 16)] = jnp.zeros((16,), jnp.int32)
            zzf_v[pl.ds(i * 16, 16)] = jnp.zeros((16,), jnp.float32)
            return 0
        lax.fori_loop(0, RPT // 16, zbody, 0)
        pltpu.sync_copy(zz_v, rt_sh.at[pl.ds(sid * RPT, RPT)])
        pltpu.sync_copy(zzf_v, ws_sh.at[pl.ds(sid * RPT, RPT)])
        pltpu.sync_copy(zzf_v, fl_sh.at[pl.ds(sid * RPT, RPT)])

        for r in range(4):
            def obody(i, _):
                one_v[r, pl.ds(i * 16, 16)] = jnp.full((16,), 1.0, jnp.float32)
                return 0
            lax.fori_loop(0, 8, obody, 0)

        plsc.subcore_barrier()

        pltpu.sync_copy(cnt_sh, cnt_v)

        def pbody(w, carry):
            tot, mybase = carry
            row = cnt_v[pl.ds(w * 16, 16)]
            mybase = mybase + row * jnp.where(w < sid, 1, 0)
            return tot + row, mybase

        tot, mybase = lax.fori_loop(
            0, TILES, pbody,
            (jnp.zeros((16,), jnp.int32), jnp.zeros((16,), jnp.int32)))
        padded = ((tot + (TBLK - 1)) >> 8) << 8
        inc = plsc.cumsum(padded)
        segstart = inc - padded
        base_v[...] = segstart + mybase
        ebk_v[...] = inc >> 8
        nblk = jnp.max(inc) >> 8

        @pl.when(sid == 0)
        def _bmap():
            ebks = [_splat(ebk_v, e) for e in range(E)]
            for c in range(NBP // 16):
                bid = _iota16() + (16 * c)
                be = jnp.zeros((16,), jnp.int32)
                for e in range(E):
                    be = be + jnp.where(bid >= ebks[e], 1, 0)
                valid = jnp.where(bid < nblk, 1, 0)
                bexp_v[pl.ds(c * 16, 16)] = be * valid
                bval_v[pl.ds(c * 16, 16)] = valid
            pltpu.sync_copy(bexp_v, bexp_hbm)
            pltpu.sync_copy(bval_v, bval_hbm)

        runs = [_splat(base_v, e) for e in range(E)]

        def dest_arr(arr_ref, dst_ref, runs):
            for r in range(4):
                def dbody(c, runs):
                    v = arr_ref[r, pl.ds(c * 16, 16)]
                    dest = jnp.zeros((16,), jnp.int32)
                    new_runs = []
                    for e in range(E):
                        m = v == e
                        rank = plsc.cumsum(jnp.where(m, 1, 0))
                        dest = jnp.where(m, runs[e] + rank - 1, dest)
                        pop = jnp.sum(jnp.where(m, 1, 0))
                        new_runs.append(runs[e] + pop)
                    dst_ref[r, pl.ds(c * 16, 16)] = dest
                    return tuple(new_runs)
                runs = lax.fori_loop(0, 8, dbody, tuple(runs))
            return runs

        runs = dest_arr(i1_v, d1_v, runs)
        runs = dest_arr(i2_v, d2_v, runs)

        for r in range(4):
            def tbody(i, _):
                tok_v[r, pl.ds(i * 16, 16)] = (
                    _iota16() + (sid * APT + r * 128 + i * 16))
                return 0
            lax.fori_loop(0, 8, tbody, 0)

        pltpu.sync_copy(d1_v, pos0_hbm.at[pl.ds(sid * 4, 4)])
        pltpu.sync_copy(d2_v, pos1_hbm.at[pl.ds(sid * 4, 4)])

        for j in range(4):
            pltpu.sync_copy(tok_v.at[j], rt_sh.at[d1_v.at[j]], add=True)
            pltpu.sync_copy(tok_v.at[j], rt_sh.at[d2_v.at[j]], add=True)
            pltpu.sync_copy(wv1_v.at[j], ws_sh.at[d1_v.at[j]], add=True)
            pltpu.sync_copy(wv2_v.at[j], ws_sh.at[d2_v.at[j]], add=True)
            pltpu.sync_copy(one_v.at[j], fl_sh.at[d1_v.at[j]], add=True)

        plsc.subcore_barrier()

        pltpu.sync_copy(rt_sh.at[pl.ds(sid * RPT, RPT)],
                        rt_hbm.at[pl.ds(sid * RPT, RPT)])
        pltpu.sync_copy(ws_sh.at[pl.ds(sid * RPT, RPT)],
                        ws_hbm.at[pl.ds(sid * RPT, RPT)])
        pltpu.sync_copy(fl_sh.at[pl.ds(sid * RPT, RPT)],
                        fl_hbm.at[pl.ds(sid * RPT, RPT)])


def _route(i1, i2, w1, w2):
    mesh = plsc.VectorSubcoreMesh(core_axis_name="c", subcore_axis_name="s")
    f = pl.kernel(
        _route_body,
        out_type=[
            jax.ShapeDtypeStruct((R,), jnp.int32),
            jax.ShapeDtypeStruct((R,), jnp.float32),
            jax.ShapeDtypeStruct((R,), jnp.float32),
            jax.ShapeDtypeStruct((NT // 128, 128), jnp.int32),
            jax.ShapeDtypeStruct((NT // 128, 128), jnp.int32),
            jax.ShapeDtypeStruct((NBP,), jnp.int32),
            jax.ShapeDtypeStruct((NBP,), jnp.int32),
        ],
        mesh=mesh,
        scratch_types=[
            pltpu.VMEM((4, 128), jnp.int32),
            pltpu.VMEM((4, 128), jnp.int32),
            pltpu.VMEM((4, 128), jnp.float32),
            pltpu.VMEM((4, 128), jnp.float32),
            pltpu.VMEM((4, 128), jnp.int32),
            pltpu.VMEM((4, 128), jnp.int32),
            pltpu.VMEM((4, 128), jnp.int32),
            pltpu.VMEM((4, 128), jnp.float32),
            pltpu.VMEM((16,), jnp.int32),
            pltpu.VMEM((256,), jnp.int32),
            pltpu.VMEM((16,), jnp.int32),
            pltpu.VMEM((16,), jnp.int32),
            pltpu.VMEM((NBP,), jnp.int32),
            pltpu.VMEM((NBP,), jnp.int32),
            pltpu.VMEM((RPT,), jnp.int32),
            pltpu.VMEM((RPT,), jnp.float32),
            pltpu.VMEM_SHARED((256,), jnp.int32),
            pltpu.VMEM_SHARED((R,), jnp.int32),
            pltpu.VMEM_SHARED((R,), jnp.float32),
            pltpu.VMEM_SHARED((R,), jnp.float32),
        ],
        compiler_params=pltpu.CompilerParams(needs_layout_passes=False),
    )
    return f(i1, i2, w1, w2)


def _gather_body(half, rt_hbm, tokens_hbm, xs_hbm, rt_v, bufa_v, bufb_v,
                 sga, sgb, soa, sob):
    cid = lax.axis_index("c")
    sid = lax.axis_index("s")
    wid = sid * 2 + cid
    base = wid * RPW
    pltpu.sync_copy(rt_hbm.at[pl.ds(half * R2 + base, RPW)], rt_v)
    bufs = (bufa_v, bufb_v)
    gsems = (sga, sgb)
    osems = (soa, sob)
    nch = RPW // GCH
    gcps = [None] * nch
    ocps = [None] * nch

    def issue(c):
        return pltpu.async_copy(
            tokens_hbm.at[rt_v.at[pl.ds(c * GCH, GCH)]],
            bufs[c % 2], gsems[c % 2])

    gcps[0] = issue(0)
    for c in range(nch):
        gcps[c].wait()
        ocps[c] = pltpu.async_copy(
            bufs[c % 2], xs_hbm.at[pl.ds(base + c * GCH, GCH)], osems[c % 2])
        if c + 1 < nch:
            if c >= 1:
                ocps[c - 1].wait()
            gcps[c + 1] = issue(c + 1)
    ocps[nch - 2].wait()
    ocps[nch - 1].wait()


def _gather(rt, tokens, half):
    mesh = plsc.VectorSubcoreMesh(core_axis_name="c", subcore_axis_name="s")
    f = pl.kernel(
        functools.partial(_gather_body, half),
        out_type=jax.ShapeDtypeStruct((R2, H), jnp.float32),
        mesh=mesh,
        scratch_types=[
            pltpu.VMEM((RPW,), jnp.int32),
            pltpu.VMEM((GCH, H), jnp.float32),
            pltpu.VMEM((GCH, H), jnp.float32),
            pltpu.SemaphoreType.DMA,
            pltpu.SemaphoreType.DMA,
            pltpu.SemaphoreType.DMA,
            pltpu.SemaphoreType.DMA,
        ],
        compiler_params=pltpu.CompilerParams(needs_layout_passes=False),
    )
    return f(rt, tokens)


def _ffn_body(nboff, bexp_ref, bval_ref, xs_ref, w1_ref, b1_ref, w2_ref,
              b2_ref, ws_ref, fl_ref, out_ref):
    b = pl.program_id(0)

    @pl.when(bval_ref[b + nboff] == 1)
    def _compute():
        x = xs_ref[...]
        h = jax.lax.dot_general(
            x, w1_ref[0], (((1,), (0,)), ((), ())),
            preferred_element_type=jnp.float32,
        ) + b1_ref[0]
        h = 0.5 * h * (1.0 + jax.lax.erf(h * 0.7071067811865476))
        y = jax.lax.dot_general(
            h, w2_ref[0], (((1,), (0,)), ((), ())),
            preferred_element_type=jnp.float32,
        ) + b2_ref[0]
        out_ref[...] = y * ws_ref[0] + x * fl_ref[0]


def _ffn(bexp, bval, xs, W1, b1, W2, b2, ws, fl, nboff):
    grid_spec = pltpu.PrefetchScalarGridSpec(
        num_scalar_prefetch=2,
        grid=(NB2,),
        in_specs=[
            pl.BlockSpec((TBLK, H), lambda b, be, bv: (b, 0)),
            pl.BlockSpec((1, H, F), lambda b, be, bv: (be[b + nboff], 0, 0)),
            pl.BlockSpec((1, 1, F), lambda b, be, bv: (be[b + nboff], 0, 0)),
            pl.BlockSpec((1, F, H), lambda b, be, bv: (be[b + nboff], 0, 0)),
            pl.BlockSpec((1, 1, H), lambda b, be, bv: (be[b + nboff], 0, 0)),
            pl.BlockSpec((1, TBLK, 1), lambda b, be, bv: (b + nboff, 0, 0)),
            pl.BlockSpec((1, TBLK, 1), lambda b, be, bv: (b + nboff, 0, 0)),
        ],
        out_specs=pl.BlockSpec((TBLK, H), lambda b, be, bv: (b, 0)),
    )
    return pl.pallas_call(
        functools.partial(_ffn_body, nboff),
        grid_spec=grid_spec,
        out_shape=jax.ShapeDtypeStruct((R2, H), jnp.float32),
    )(bexp, bval, xs, W1, b1.reshape(E, 1, F), W2, b2.reshape(E, 1, H),
      ws.reshape(NB, TBLK, 1), fl.reshape(NB, TBLK, 1))


def _combine_body(yl_hbm, yh_hbm, pos0_hbm, pos1_hbm, out_hbm,
                  p0_v, p1_v, q0lo_v, q0hi_v, q1lo_v, q1hi_v,
                  b0lo_a, b0lo_b, b0hi_a, b0hi_b,
                  b1lo_a, b1lo_b, b1hi_a, b1hi_b, outb_v,
                  s0la, s0lb, s0ha, s0hb, s1la, s1lb, s1ha, s1hb):
    cid = lax.axis_index("c")
    sid = lax.axis_index("s")
    wid = sid * 2 + cid
    pltpu.sync_copy(pos0_hbm.at[pl.ds(wid * 2, 2)], p0_v)
    pltpu.sync_copy(pos1_hbm.at[pl.ds(wid * 2, 2)], p1_v)

    for j in range(2):
        def cbody(k, _):
            sl = pl.ds(k * 16, 16)
            v0 = p0_v[j, sl]
            q0lo_v[j, sl] = jnp.minimum(v0, R2 - 1)
            q0hi_v[j, sl] = jnp.maximum(v0 - R2, 0)
            v1 = p1_v[j, sl]
            q1lo_v[j, sl] = jnp.minimum(v1, R2 - 1)
            q1hi_v[j, sl] = jnp.maximum(v1 - R2, 0)
            return 0
        lax.fori_loop(0, 8, cbody, 0)

    b0los = (b0lo_a, b0lo_b)
    b0his = (b0hi_a, b0hi_b)
    b1los = (b1lo_a, b1lo_b)
    b1his = (b1hi_a, b1hi_b)
    sems = ((s0la, s0ha, s1la, s1ha), (s0lb, s0hb, s1lb, s1hb))
    nch = TPW // CCH
    cps = [None] * nch

    def issue(c):
        j, col = divmod(c * CCH, 128)
        sl = pl.ds(col, CCH)
        sm = sems[c % 2]
        return (
            pltpu.async_copy(yl_hbm.at[q0lo_v.at[j, sl]], b0los[c % 2], sm[0]),
            pltpu.async_copy(yh_hbm.at[q0hi_v.at[j, sl]], b0his[c % 2], sm[1]),
            pltpu.async_copy(yl_hbm.at[q1lo_v.at[j, sl]], b1los[c % 2], sm[2]),
            pltpu.async_copy(yh_hbm.at[q1hi_v.at[j, sl]], b1his[c % 2], sm[3]),
        )

    cps[0] = issue(0)
    for c in range(nch):
        for cp in cps[c]:
            cp.wait()
        if c + 1 < nch:
            cps[c + 1] = issue(c + 1)
        a0lo, a0hi = b0los[c % 2], b0his[c % 2]
        a1lo, a1hi = b1los[c % 2], b1his[c % 2]
        for r in range(CCH):
            jr, cr = divmod(c * CCH + r, 128)
            jv = jnp.full((16,), jr, jnp.int32)
            cv = jnp.full((16,), cr, jnp.int32)
            m0 = plsc.load_gather(p0_v, [jv, cv]) < R2
            m1 = plsc.load_gather(p1_v, [jv, cv]) < R2

            def abody(k, _):
                sl = pl.ds(k * 16, 16)
                y0 = jnp.where(m0, a0lo[r, sl], a0hi[r, sl])
                y1 = jnp.where(m1, a1lo[r, sl], a1hi[r, sl])
                outb_v[r, sl] = y0 + y1
                return 0
            lax.fori_loop(0, H // 16, abody, 0)
        pltpu.sync_copy(outb_v, out_hbm.at[pl.ds(wid * TPW + c * CCH, CCH)])


def _combine(ys_lo, ys_hi, pos0, pos1):
    mesh = plsc.VectorSubcoreMesh(core_axis_name="c", subcore_axis_name="s")
    f = pl.kernel(
        _combine_body,
        out_type=jax.ShapeDtypeStruct((NT, H), jnp.float32),
        mesh=mesh,
        scratch_types=[
            pltpu.VMEM((2, 128), jnp.int32),
            pltpu.VMEM((2, 128), jnp.int32),
            pltpu.VMEM((2, 128), jnp.int32),
            pltpu.VMEM((2, 128), jnp.int32),
            pltpu.VMEM((2, 128), jnp.int32),
            pltpu.VMEM((2, 128), jnp.int32),
            pltpu.VMEM((CCH, H), jnp.float32),
            pltpu.VMEM((CCH, H), jnp.float32),
            pltpu.VMEM((CCH, H), jnp.float32),
            pltpu.VMEM((CCH, H), jnp.float32),
            pltpu.VMEM((CCH, H), jnp.float32),
            pltpu.VMEM((CCH, H), jnp.float32),
            pltpu.VMEM((CCH, H), jnp.float32),
            pltpu.VMEM((CCH, H), jnp.float32),
            pltpu.VMEM((CCH, H), jnp.float32),
            pltpu.SemaphoreType.DMA,
            pltpu.SemaphoreType.DMA,
            pltpu.SemaphoreType.DMA,
            pltpu.SemaphoreType.DMA,
            pltpu.SemaphoreType.DMA,
            pltpu.SemaphoreType.DMA,
            pltpu.SemaphoreType.DMA,
            pltpu.SemaphoreType.DMA,
        ],
        compiler_params=pltpu.CompilerParams(needs_layout_passes=False),
    )
    return f(ys_lo, ys_hi, pos0, pos1)


def kernel(hidden_states, Wg, W1, b1, W2, b2):
    Bv, Sv, Hv = hidden_states.shape
    tokens = hidden_states.reshape(NT, Hv)
    i1, i2, w1, w2 = _gate(tokens, Wg)
    rt, ws, fl, pos0, pos1, bexp, bval = _route(i1, i2, w1, w2)
    xs_lo = _gather(rt, tokens, 0)
    xs_hi = _gather(rt, tokens, 1)
    ys_lo = _ffn(bexp, bval, xs_lo, W1, b1, W2, b2, ws, fl, 0)
    ys_hi = _ffn(bexp, bval, xs_hi, W1, b1, W2, b2, ws, fl, NB2)
    out = _combine(ys_lo, ys_hi, pos0, pos1)
    return out.reshape(Bv, Sv, Hv)


# speedup vs baseline: 1.6322x; 1.6322x over previous
import functools

import jax
import jax.numpy as jnp
from jax import lax
from jax.experimental import pallas as pl
from jax.experimental.pallas import tpu as pltpu
from jax.experimental.pallas import tpu_sc as plsc

H = 768
F = 3072
E = 8
NT = 8192
TBLK = 256
NB = 72
NBP = 80
R = NB * TBLK
TILES = 16
APT = NT // TILES
RPT = R // TILES
W32 = 32
RPW = R // W32
GCH = 64
TPW = NT // W32
CCH = 32


def _gate_body(x_ref, wg_ref, i1_ref, i2_ref, w1_ref, w2_ref):
    x = x_ref[...]
    logits = jax.lax.dot_general(
        x, wg_ref[...], (((1,), (1,)), ((), ())),
        preferred_element_type=jnp.float32,
    )
    probs = jax.nn.softmax(logits, axis=-1)
    eidx = jax.lax.broadcasted_iota(jnp.int32, probs.shape, 1)
    p1 = jnp.max(probs, axis=-1, keepdims=True)
    first1 = jnp.min(jnp.where(probs == p1, eidx, E), axis=-1, keepdims=True)
    m1 = eidx == first1
    probsb = jnp.where(m1, -jnp.inf, probs)
    p2 = jnp.max(probsb, axis=-1, keepdims=True)
    first2 = jnp.min(jnp.where(probsb == p2, eidx, E), axis=-1, keepdims=True)
    denom = p1 + p2
    i1_ref[...] = first1.astype(jnp.int32).reshape(1, 4, 128)
    i2_ref[...] = first2.astype(jnp.int32).reshape(1, 4, 128)
    w1_ref[...] = (p1 / denom).reshape(1, 4, 128)
    w2_ref[...] = (p2 / denom).reshape(1, 4, 128)


def _gate(tokens, Wg):
    return pl.pallas_call(
        _gate_body,
        grid=(NT // 512,),
        in_specs=[
            pl.BlockSpec((512, H), lambda t: (t, 0)),
            pl.BlockSpec((E, H), lambda t: (0, 0)),
        ],
        out_specs=[
            pl.BlockSpec((1, 4, 128), lambda t: (t, 0, 0)),
            pl.BlockSpec((1, 4, 128), lambda t: (t, 0, 0)),
            pl.BlockSpec((1, 4, 128), lambda t: (t, 0, 0)),
            pl.BlockSpec((1, 4, 128), lambda t: (t, 0, 0)),
        ],
        out_shape=[
            jax.ShapeDtypeStruct((16, 4, 128), jnp.int32),
            jax.ShapeDtypeStruct((16, 4, 128), jnp.int32),
            jax.ShapeDtypeStruct((16, 4, 128), jnp.float32),
            jax.ShapeDtypeStruct((16, 4, 128), jnp.float32),
        ],
    )(tokens, Wg)


def _iota16():
    return jax.lax.iota(jnp.int32, 16)


def _splat(vmem_ref, lane):
    return plsc.load_gather(vmem_ref, [jnp.full((16,), lane, jnp.int32)])


def _route_body(i1_hbm, i2_hbm, w1_hbm, w2_hbm,
                rt_hbm, ws_hbm, fl_hbm, pos0_hbm, pos1_hbm, bexp_hbm, bval_hbm,
                i1_v, i2_v, wv1_v, wv2_v, d1_v, d2_v, tok_v, one_v,
                cntrow_v, cnt_v, ebk_v, base_v, bexp_v, bval_v,
                zz_v, zzf_v,
                cnt_sh, rt_sh, ws_sh, fl_sh):
    cid = lax.axis_index("c")
    sid = lax.axis_index("s")

    @pl.when(cid == 0)
    def _core0():
        pltpu.sync_copy(i1_hbm.at[sid], i1_v)
        pltpu.sync_copy(i2_hbm.at[sid], i2_v)
        pltpu.sync_copy(w1_hbm.at[sid], wv1_v)
        pltpu.sync_copy(w2_hbm.at[sid], wv2_v)

        def hist_arr(arr_ref, acc):
            for r in range(4):
                def hbody(c, acc):
                    v = arr_ref[r, pl.ds(c * 16, 16)]
                    for e in range(E):
                        pop = jnp.sum(jnp.where(v == e, 1, 0))
                        acc = acc + jnp.where(_iota16() == e, pop, 0)
                    return acc
                acc = lax.fori_loop(0, 8, hbody, acc)
            return acc

        cnt = jnp.zeros((16,), jnp.int32)
        cnt = hist_arr(i1_v, cnt)
        cnt = hist_arr(i2_v, cnt)
        cntrow_v[...] = cnt
        pltpu.sync_copy(cntrow_v, cnt_sh.at[pl.ds(sid * 16, 16)])

        def zbody(i, _):
            zz_v[pl.ds(i * 16, 16)] = jnp.zeros((16,), jnp.int32)
            zzf_v[pl.ds(i * 16, 16)] = jnp.zeros((16,), jnp.float32)
            return 0
        lax.fori_loop(0, RPT // 16, zbody, 0)
        pltpu.sync_copy(zz_v, rt_sh.at[pl.ds(sid * RPT, RPT)])
        pltpu.sync_copy(zzf_v, ws_sh.at[pl.ds(sid * RPT, RPT)])
        pltpu.sync_copy(zzf_v, fl_sh.at[pl.ds(sid * RPT, RPT)])

        for r in range(4):
            def obody(i, _):
                one_v[r, pl.ds(i * 16, 16)] = jnp.full((16,), 1.0, jnp.float32)
                return 0
            lax.fori_loop(0, 8, obody, 0)

        plsc.subcore_barrier()

        pltpu.sync_copy(cnt_sh, cnt_v)

        def pbody(w, carry):
            tot, mybase = carry
            row = cnt_v[pl.ds(w * 16, 16)]
            mybase = mybase + row * jnp.where(w < sid, 1, 0)
            return tot + row, mybase

        tot, mybase = lax.fori_loop(
            0, TILES, pbody,
            (jnp.zeros((16,), jnp.int32), jnp.zeros((16,), jnp.int32)))
        padded = ((tot + (TBLK - 1)) >> 8) << 8
        inc = plsc.cumsum(padded)
        segstart = inc - padded
        base_v[...] = segstart + mybase
        ebk_v[...] = inc >> 8
        nblk = jnp.max(inc) >> 8

        @pl.when(sid == 0)
        def _bmap():
            ebks = [_splat(ebk_v, e) for e in range(E)]
            for c in range(NBP // 16):
                bid = _iota16() + (16 * c)
                be = jnp.zeros((16,), jnp.int32)
                for e in range(E):
                    be = be + jnp.where(bid >= ebks[e], 1, 0)
                valid = jnp.where(bid < nblk, 1, 0)
                bexp_v[pl.ds(c * 16, 16)] = be * valid
                bval_v[pl.ds(c * 16, 16)] = valid
            pltpu.sync_copy(bexp_v, bexp_hbm)
            pltpu.sync_copy(bval_v, bval_hbm)

        runs = [_splat(base_v, e) for e in range(E)]

        def dest_arr(arr_ref, dst_ref, runs):
            for r in range(4):
                def dbody(c, runs):
                    v = arr_ref[r, pl.ds(c * 16, 16)]
                    dest = jnp.zeros((16,), jnp.int32)
                    new_runs = []
                    for e in range(E):
                        m = v == e
                        rank = plsc.cumsum(jnp.where(m, 1, 0))
                        dest = jnp.where(m, runs[e] + rank - 1, dest)
                        pop = jnp.sum(jnp.where(m, 1, 0))
                        new_runs.append(runs[e] + pop)
                    dst_ref[r, pl.ds(c * 16, 16)] = dest
                    return tuple(new_runs)
                runs = lax.fori_loop(0, 8, dbody, tuple(runs))
            return runs

        runs = dest_arr(i1_v, d1_v, runs)
        runs = dest_arr(i2_v, d2_v, runs)

        for r in range(4):
            def tbody(i, _):
                tok_v[r, pl.ds(i * 16, 16)] = (
                    _iota16() + (sid * APT + r * 128 + i * 16))
                return 0
            lax.fori_loop(0, 8, tbody, 0)

        pltpu.sync_copy(d1_v, pos0_hbm.at[pl.ds(sid * 4, 4)])
        pltpu.sync_copy(d2_v, pos1_hbm.at[pl.ds(sid * 4, 4)])

        for j in range(4):
            pltpu.sync_copy(tok_v.at[j], rt_sh.at[d1_v.at[j]], add=True)
            pltpu.sync_copy(tok_v.at[j], rt_sh.at[d2_v.at[j]], add=True)
            pltpu.sync_copy(wv1_v.at[j], ws_sh.at[d1_v.at[j]], add=True)
            pltpu.sync_copy(wv2_v.at[j], ws_sh.at[d2_v.at[j]], add=True)
            pltpu.sync_copy(one_v.at[j], fl_sh.at[d1_v.at[j]], add=True)

        plsc.subcore_barrier()

        pltpu.sync_copy(rt_sh.at[pl.ds(sid * RPT, RPT)],
                        rt_hbm.at[pl.ds(sid * RPT, RPT)])
        pltpu.sync_copy(ws_sh.at[pl.ds(sid * RPT, RPT)],
                        ws_hbm.at[pl.ds(sid * RPT, RPT)])
        pltpu.sync_copy(fl_sh.at[pl.ds(sid * RPT, RPT)],
                        fl_hbm.at[pl.ds(sid * RPT, RPT)])


def _route(i1, i2, w1, w2):
    mesh = plsc.VectorSubcoreMesh(core_axis_name="c", subcore_axis_name="s")
    f = pl.kernel(
        _route_body,
        out_type=[
            jax.ShapeDtypeStruct((R,), jnp.int32),
            jax.ShapeDtypeStruct((R,), jnp.float32),
            jax.ShapeDtypeStruct((R,), jnp.float32),
            jax.ShapeDtypeStruct((NT // 128, 128), jnp.int32),
            jax.ShapeDtypeStruct((NT // 128, 128), jnp.int32),
            jax.ShapeDtypeStruct((NBP,), jnp.int32),
            jax.ShapeDtypeStruct((NBP,), jnp.int32),
        ],
        mesh=mesh,
        scratch_types=[
            pltpu.VMEM((4, 128), jnp.int32),
            pltpu.VMEM((4, 128), jnp.int32),
            pltpu.VMEM((4, 128), jnp.float32),
            pltpu.VMEM((4, 128), jnp.float32),
            pltpu.VMEM((4, 128), jnp.int32),
            pltpu.VMEM((4, 128), jnp.int32),
            pltpu.VMEM((4, 128), jnp.int32),
            pltpu.VMEM((4, 128), jnp.float32),
            pltpu.VMEM((16,), jnp.int32),
            pltpu.VMEM((256,), jnp.int32),
            pltpu.VMEM((16,), jnp.int32),
            pltpu.VMEM((16,), jnp.int32),
            pltpu.VMEM((NBP,), jnp.int32),
            pltpu.VMEM((NBP,), jnp.int32),
            pltpu.VMEM((RPT,), jnp.int32),
            pltpu.VMEM((RPT,), jnp.float32),
            pltpu.VMEM_SHARED((256,), jnp.int32),
            pltpu.VMEM_SHARED((R,), jnp.int32),
            pltpu.VMEM_SHARED((R,), jnp.float32),
            pltpu.VMEM_SHARED((R,), jnp.float32),
        ],
        compiler_params=pltpu.CompilerParams(needs_layout_passes=False),
    )
    return f(i1, i2, w1, w2)


def _gather_body(rt_hbm, tokens_hbm, xs_hbm, rt_v, bufa_v, bufb_v,
                 sga, sgb, soa, sob):
    cid = lax.axis_index("c")
    sid = lax.axis_index("s")
    wid = sid * 2 + cid
    base = wid * RPW
    pltpu.sync_copy(rt_hbm.at[pl.ds(base, RPW)], rt_v)
    bufs = (bufa_v, bufb_v)
    gsems = (sga, sgb)
    osems = (soa, sob)
    nch = RPW // GCH
    gcps = [None] * nch
    ocps = [None] * nch

    def issue(c):
        return pltpu.async_copy(
            tokens_hbm.at[rt_v.at[pl.ds(c * GCH, GCH)]],
            bufs[c % 2], gsems[c % 2])

    gcps[0] = issue(0)
    for c in range(nch):
        gcps[c].wait()
        ocps[c] = pltpu.async_copy(
            bufs[c % 2], xs_hbm.at[pl.ds(base + c * GCH, GCH)], osems[c % 2])
        if c + 1 < nch:
            if c >= 1:
                ocps[c - 1].wait()
            gcps[c + 1] = issue(c + 1)
    ocps[nch - 2].wait()
    ocps[nch - 1].wait()


def _gather(rt, tokens):
    mesh = plsc.VectorSubcoreMesh(core_axis_name="c", subcore_axis_name="s")
    f = pl.kernel(
        _gather_body,
        out_type=jax.ShapeDtypeStruct((R, H), jnp.float32),
        mesh=mesh,
        scratch_types=[
            pltpu.VMEM((RPW,), jnp.int32),
            pltpu.VMEM((GCH, H), jnp.float32),
            pltpu.VMEM((GCH, H), jnp.float32),
            pltpu.SemaphoreType.DMA,
            pltpu.SemaphoreType.DMA,
            pltpu.SemaphoreType.DMA,
            pltpu.SemaphoreType.DMA,
        ],
        compiler_params=pltpu.CompilerParams(needs_layout_passes=False),
    )
    return f(rt, tokens)


def _ffn_body(bexp_ref, bval_ref, xs_ref, w1_ref, b1_ref, w2_ref, b2_ref,
              ws_ref, fl_ref, out_ref):
    b = pl.program_id(0)

    @pl.when(bval_ref[b] == 1)
    def _compute():
        x = xs_ref[...]
        h = jax.lax.dot_general(
            x, w1_ref[0], (((1,), (0,)), ((), ())),
            preferred_element_type=jnp.float32,
        ) + b1_ref[0]
        h = 0.5 * h * (1.0 + jax.lax.erf(h * 0.7071067811865476))
        y = jax.lax.dot_general(
            h, w2_ref[0], (((1,), (0,)), ((), ())),
            preferred_element_type=jnp.float32,
        ) + b2_ref[0]
        out_ref[...] = y * ws_ref[0] + x * fl_ref[0]


def _ffn(bexp, bval, xs, W1, b1, W2, b2, ws, fl):
    grid_spec = pltpu.PrefetchScalarGridSpec(
        num_scalar_prefetch=2,
        grid=(NB,),
        in_specs=[
            pl.BlockSpec((TBLK, H), lambda b, be, bv: (b, 0)),
            pl.BlockSpec((1, H, F), lambda b, be, bv: (be[b], 0, 0)),
            pl.BlockSpec((1, 1, F), lambda b, be, bv: (be[b], 0, 0)),
            pl.BlockSpec((1, F, H), lambda b, be, bv: (be[b], 0, 0)),
            pl.BlockSpec((1, 1, H), lambda b, be, bv: (be[b], 0, 0)),
            pl.BlockSpec((1, TBLK, 1), lambda b, be, bv: (b, 0, 0)),
            pl.BlockSpec((1, TBLK, 1), lambda b, be, bv: (b, 0, 0)),
        ],
        out_specs=pl.BlockSpec((TBLK, H), lambda b, be, bv: (b, 0)),
    )
    return pl.pallas_call(
        _ffn_body,
        grid_spec=grid_spec,
        out_shape=jax.ShapeDtypeStruct((R, H), jnp.float32),
    )(bexp, bval, xs, W1, b1.reshape(E, 1, F), W2, b2.reshape(E, 1, H),
      ws.reshape(NB, TBLK, 1), fl.reshape(NB, TBLK, 1))


def _combine_body(ys_hbm, pos0_hbm, pos1_hbm, out_hbm,
                  p0_v, p1_v, b0a_v, b0b_v, b1a_v, b1b_v, outb_v,
                  s0a, s0b, s1a, s1b):
    cid = lax.axis_index("c")
    sid = lax.axis_index("s")
    wid = sid * 2 + cid
    pltpu.sync_copy(pos0_hbm.at[pl.ds(wid * 2, 2)], p0_v)
    pltpu.sync_copy(pos1_hbm.at[pl.ds(wid * 2, 2)], p1_v)
    b0s = (b0a_v, b0b_v)
    b1s = (b1a_v, b1b_v)
    s0s = (s0a, s0b)
    s1s = (s1a, s1b)
    nch = TPW // CCH
    cps = [None] * nch

    def issue(c):
        j, col = divmod(c * CCH, 128)
        cp0 = pltpu.async_copy(
            ys_hbm.at[p0_v.at[j, pl.ds(col, CCH)]], b0s[c % 2], s0s[c % 2])
        cp1 = pltpu.async_copy(
            ys_hbm.at[p1_v.at[j, pl.ds(col, CCH)]], b1s[c % 2], s1s[c % 2])
        return cp0, cp1

    cps[0] = issue(0)
    for c in range(nch):
        cps[c][0].wait()
        cps[c][1].wait()
        if c + 1 < nch:
            cps[c + 1] = issue(c + 1)
        a = b0s[c % 2]
        b = b1s[c % 2]

        def abody(r, _):
            for k in range(H // 16):
                s = pl.ds(k * 16, 16)
                outb_v[r, s] = a[r, s] + b[r, s]
            return 0
        lax.fori_loop(0, CCH, abody, 0)
        pltpu.sync_copy(outb_v, out_hbm.at[pl.ds(wid * TPW + c * CCH, CCH)])


def _combine(ys, pos0, pos1):
    mesh = plsc.VectorSubcoreMesh(core_axis_name="c", subcore_axis_name="s")
    f = pl.kernel(
        _combine_body,
        out_type=jax.ShapeDtypeStruct((NT, H), jnp.float32),
        mesh=mesh,
        scratch_types=[
            pltpu.VMEM((2, 128), jnp.int32),
            pltpu.VMEM((2, 128), jnp.int32),
            pltpu.VMEM((CCH, H), jnp.float32),
            pltpu.VMEM((CCH, H), jnp.float32),
            pltpu.VMEM((CCH, H), jnp.float32),
            pltpu.VMEM((CCH, H), jnp.float32),
            pltpu.VMEM((CCH, H), jnp.float32),
            pltpu.SemaphoreType.DMA,
            pltpu.SemaphoreType.DMA,
            pltpu.SemaphoreType.DMA,
            pltpu.SemaphoreType.DMA,
        ],
        compiler_params=pltpu.CompilerParams(needs_layout_passes=False),
    )
    return f(ys, pos0, pos1)


def kernel(hidden_states, Wg, W1, b1, W2, b2):
    Bv, Sv, Hv = hidden_states.shape
    tokens = hidden_states.reshape(NT, Hv)
    i1, i2, w1, w2 = _gate(tokens, Wg)
    rt, ws, fl, pos0, pos1, bexp, bval = _route(i1, i2, w1, w2)
    xs = _gather(rt, tokens)
    ys = _ffn(bexp, bval, xs, W1, b1, W2, b2, ws, fl)
    out = _combine(ys, pos0, pos1)
    return out.reshape(Bv, Sv, Hv)
